# baseline (speedup 1.0000x reference)
import functools

import jax
import jax.numpy as jnp
from jax.experimental import pallas as pl

N_ATOMS = 50000
N_BONDS = 800000
ATOM_FDIM = 133
BOND_FDIM = 13
HIDDEN = 64
TASKS = 12
DEPTH = 3
N_MOLS = 500


def _mm_body(x_ref, w_ref, o_ref, *, relu):
    y = jnp.dot(x_ref[...], w_ref[...], preferred_element_type=jnp.float32)
    o_ref[...] = jnp.maximum(y, 0.0) if relu else y


def _mm_add_body(x_ref, w_ref, a_ref, o_ref, *, relu):
    y = jnp.dot(x_ref[...], w_ref[...], preferred_element_type=jnp.float32)
    y = y + a_ref[...]
    o_ref[...] = jnp.maximum(y, 0.0) if relu else y


def _mm(x, w, add=None, relu=False, blk=8000):
    n, k = x.shape
    h = w.shape[1]
    assert n % blk == 0, (n, blk)
    grid = (n // blk,)
    in_specs = [
        pl.BlockSpec((blk, k), lambda i: (i, 0)),
        pl.BlockSpec((k, h), lambda i: (0, 0)),
    ]
    args = [x, w]
    if add is not None:
        in_specs.append(pl.BlockSpec((blk, h), lambda i: (i, 0)))
        args.append(add)
        body = functools.partial(_mm_add_body, relu=relu)
    else:
        body = functools.partial(_mm_body, relu=relu)
    return pl.pallas_call(
        body,
        grid=grid,
        in_specs=in_specs,
        out_specs=pl.BlockSpec((blk, h), lambda i: (i, 0)),
        out_shape=jax.ShapeDtypeStruct((n, h), jnp.float32),
    )(*args)


def _combine_body(h0_ref, g1_ref, g2_ref, o_ref):
    o_ref[...] = jnp.maximum(h0_ref[...] + g1_ref[...] - g2_ref[...], 0.0)


def _combine(h0, g1, g2, blk=8000):
    n, h = h0.shape
    spec = pl.BlockSpec((blk, h), lambda i: (i, 0))
    return pl.pallas_call(
        _combine_body,
        grid=(n // blk,),
        in_specs=[spec, spec, spec],
        out_specs=spec,
        out_shape=jax.ShapeDtypeStruct((n, h), jnp.float32),
    )(h0, g1, g2)


def _head_body(mv_ref, r1w_ref, r1b_ref, r2w_ref, r2b_ref, o_ref):
    out = jnp.maximum(
        jnp.dot(mv_ref[...], r1w_ref[...], preferred_element_type=jnp.float32)
        + r1b_ref[...], 0.0)
    o_ref[...] = (
        jnp.dot(out, r2w_ref[...], preferred_element_type=jnp.float32)
        + r2b_ref[...])


def _head(mol_vecs, R1_t, R1_b, R2_t, R2_b):
    m = mol_vecs.shape[0]
    return pl.pallas_call(
        _head_body,
        out_shape=jax.ShapeDtypeStruct((m, TASKS), jnp.float32),
    )(mol_vecs, R1_t, R1_b.reshape(1, -1), R2_t, R2_b.reshape(1, -1))


def kernel(f_atoms, f_bonds, b2a, b2revb, mol_ids, W_i, W_h, W_o_w, W_o_b,
           R1_w, R1_b, R2_w, R2_b):
    W_ia_t = W_i[:, :ATOM_FDIM].T
    W_ib_t = W_i[:, ATOM_FDIM:].T
    W_h_t = W_h.T
    W_oa_t = W_o_w[:, :ATOM_FDIM].T
    W_om_t = W_o_w[:, ATOM_FDIM:].T

    A0 = _mm(f_atoms, W_ia_t, blk=2000)
    G = jnp.take(A0, b2a, axis=0)
    h0 = _mm(f_bonds, W_ib_t, add=G, relu=True)

    target_atoms = jnp.take(b2a, b2revb, axis=0)
    h = h0
    for _ in range(DEPTH):
        S = jax.ops.segment_sum(h, target_atoms, num_segments=N_ATOMS)
        AM2 = _mm(S, W_h_t, blk=2000)
        Hh = _mm(h, W_h_t)
        g1 = jnp.take(AM2, b2a, axis=0)
        g2 = jnp.take(Hh, b2revb, axis=0)
        h = _combine(h0, g1, g2)

    S = jax.ops.segment_sum(h, target_atoms, num_segments=N_ATOMS)
    A1 = _mm(f_atoms, W_oa_t, blk=2000)
    SB = _mm(S, W_om_t, add=A1 + W_o_b.reshape(1, -1), relu=True, blk=2000)
    mol_vecs = jax.ops.segment_sum(SB, mol_ids, num_segments=N_MOLS)
    return _head(mol_vecs, R1_w.T, R1_b, R2_w.T, R2_b)
